# Initial kernel scaffold; baseline (speedup 1.0000x reference)
import jax
import jax.numpy as jnp
from jax.experimental import pallas as pl


def kernel(z_in, z_out, z_self, edge_index, W_in, b_in, W_out, b_out):
    raise NotImplementedError("write your pallas kernel here")


# speedup vs baseline: 1.3582x; 1.3582x over previous
import jax
import jax.numpy as jnp
from jax import lax
from jax.experimental import pallas as pl
from jax.experimental.pallas import tpu as pltpu
from jax.experimental.pallas import tpu_sc as plsc

N = 10000
D = 256
E = 160000

NC = 2
NS = 16
NW = NC * NS
LANES = 16
DD = 2 * D
G = 313
EPW = G * LANES
E_PAD = NW * EPW

ROWS_TC = 1000


def _tc_pack_body(zs_ref, zo_ref, zi_ref, wi_ref, bi_ref, wo_ref, bo_ref,
                  p_ref, q_ref):
    zs = zs_ref[...]
    a_in = lax.dot_general(zs, wi_ref[...], (((1,), (1,)), ((), ())),
                           preferred_element_type=jnp.float32) + bi_ref[...]
    a_out = lax.dot_general(zs, wo_ref[...], (((1,), (1,)), ((), ())),
                            preferred_element_type=jnp.float32) + bo_ref[...]
    p_ref[:, :D] = zo_ref[...]
    p_ref[:, D:] = a_in
    q_ref[:, :D] = a_out
    q_ref[:, D:] = zi_ref[...]


def _build_pq(z_in, z_out, z_self, W_in, b_in, W_out, b_out):
    b_in2 = b_in.reshape(1, D)
    b_out2 = b_out.reshape(1, D)
    grid = (N // ROWS_TC,)
    row_spec = pl.BlockSpec((ROWS_TC, D), lambda i: (i, 0))
    full_spec = pl.BlockSpec((D, D), lambda i: (0, 0))
    bias_spec = pl.BlockSpec((1, D), lambda i: (0, 0))
    out_spec = pl.BlockSpec((ROWS_TC, DD), lambda i: (i, 0))
    return pl.pallas_call(
        _tc_pack_body,
        grid=grid,
        in_specs=[row_spec, row_spec, row_spec, full_spec, bias_spec,
                  full_spec, bias_spec],
        out_specs=[out_spec, out_spec],
        out_shape=[jax.ShapeDtypeStruct((N, DD), jnp.float32),
                   jax.ShapeDtypeStruct((N, DD), jnp.float32)],
    )(z_self, z_out, z_in, W_in, b_in2, W_out, b_out2)


def _sc_edge_body(p_hbm, q_hbm, src_hbm, dst_hbm, out_hbm,
                  src_all, dst_all, out_all, p_rows, q_rows, accbuf,
                  sem_p, sem_q):
    wid = lax.axis_index("s") * NC + lax.axis_index("c")
    base = wid * EPW
    pltpu.sync_copy(src_hbm.at[pl.ds(base, EPW)], src_all)
    pltpu.sync_copy(dst_hbm.at[pl.ds(base, EPW)], dst_all)
    lane16 = lax.iota(jnp.int32, LANES) * LANES

    def group(g, carry):
        src16 = src_all[pl.ds(g * LANES, LANES)]
        dst16 = dst_all[pl.ds(g * LANES, LANES)]
        cp = pltpu.async_copy(p_hbm.at[src16], p_rows, sem_p)
        cq = pltpu.async_copy(q_hbm.at[dst16], q_rows, sem_q)
        cp.wait()
        cq.wait()
        for e in range(LANES):
            acc = p_rows[e, pl.ds(0, LANES)] * q_rows[e, pl.ds(0, LANES)]
            for j in range(1, DD // LANES):
                acc = acc + (p_rows[e, pl.ds(j * LANES, LANES)]
                             * q_rows[e, pl.ds(j * LANES, LANES)])
            accbuf[pl.ds(e * LANES, LANES)] = acc
        totals = plsc.load_gather(accbuf, [lane16])
        for j in range(1, LANES):
            totals = totals + plsc.load_gather(accbuf, [lane16 + j])
        out_all[pl.ds(g * LANES, LANES)] = 1.0 / (1.0 + jnp.exp(-0.5 * totals))
        return carry

    lax.fori_loop(0, G, group, 0)
    pltpu.sync_copy(out_all, out_hbm.at[pl.ds(base, EPW)])


def _edge_scores(P, Q, src, dst):
    mesh = plsc.VectorSubcoreMesh(core_axis_name="c", subcore_axis_name="s",
                                  num_cores=NC, num_subcores=NS)
    run = pl.kernel(
        _sc_edge_body,
        out_type=jax.ShapeDtypeStruct((E_PAD,), jnp.float32),
        mesh=mesh,
        compiler_params=pltpu.CompilerParams(needs_layout_passes=False),
        scratch_types=[
            pltpu.VMEM((EPW,), jnp.int32),
            pltpu.VMEM((EPW,), jnp.int32),
            pltpu.VMEM((EPW,), jnp.float32),
            pltpu.VMEM((LANES, DD), jnp.float32),
            pltpu.VMEM((LANES, DD), jnp.float32),
            pltpu.VMEM((LANES * LANES,), jnp.float32),
            pltpu.SemaphoreType.DMA,
            pltpu.SemaphoreType.DMA,
        ],
    )
    return run(P, Q, src, dst)


def kernel(z_in, z_out, z_self, edge_index, W_in, b_in, W_out, b_out):
    P, Q = _build_pq(z_in, z_out, z_self, W_in, b_in, W_out, b_out)
    src = jnp.pad(edge_index[0].astype(jnp.int32), (0, E_PAD - E))
    dst = jnp.pad(edge_index[1].astype(jnp.int32), (0, E_PAD - E))
    return _edge_scores(P, Q, src, dst)[:E]


# speedup vs baseline: 1.9488x; 1.4348x over previous
import jax
import jax.numpy as jnp
from jax import lax
from jax.experimental import pallas as pl
from jax.experimental.pallas import tpu as pltpu
from jax.experimental.pallas import tpu_sc as plsc

N = 10000
D = 256
E = 160000

NC = 2
NS = 16
NW = NC * NS
LANES = 16
DD = 2 * D
G = 314
EPW = G * LANES
E_PAD = NW * EPW

ROWS_TC = 1000


def _tc_pack_body(zs_ref, zo_ref, zi_ref, wi_ref, bi_ref, wo_ref, bo_ref,
                  p_ref, q_ref):
    zs = zs_ref[...]
    a_in = lax.dot_general(zs, wi_ref[...], (((1,), (1,)), ((), ())),
                           preferred_element_type=jnp.float32) + bi_ref[...]
    a_out = lax.dot_general(zs, wo_ref[...], (((1,), (1,)), ((), ())),
                            preferred_element_type=jnp.float32) + bo_ref[...]
    p_ref[:, :D] = zo_ref[...]
    p_ref[:, D:] = a_in
    q_ref[:, :D] = a_out
    q_ref[:, D:] = zi_ref[...]


def _build_pq(z_in, z_out, z_self, W_in, b_in, W_out, b_out):
    b_in2 = b_in.reshape(1, D)
    b_out2 = b_out.reshape(1, D)
    grid = (N // ROWS_TC,)
    row_spec = pl.BlockSpec((ROWS_TC, D), lambda i: (i, 0))
    full_spec = pl.BlockSpec((D, D), lambda i: (0, 0))
    bias_spec = pl.BlockSpec((1, D), lambda i: (0, 0))
    out_spec = pl.BlockSpec((ROWS_TC, DD), lambda i: (i, 0))
    return pl.pallas_call(
        _tc_pack_body,
        grid=grid,
        in_specs=[row_spec, row_spec, row_spec, full_spec, bias_spec,
                  full_spec, bias_spec],
        out_specs=[out_spec, out_spec],
        out_shape=[jax.ShapeDtypeStruct((N, DD), jnp.float32),
                   jax.ShapeDtypeStruct((N, DD), jnp.float32)],
    )(z_self, z_out, z_in, W_in, b_in2, W_out, b_out2)


def _sc_edge_body(p_hbm, q_hbm, src_hbm, dst_hbm, out_hbm,
                  src_all, dst_all, out_all, p_rows, q_rows, accbuf,
                  sem_p0, sem_q0, sem_p1, sem_q1):
    wid = lax.axis_index("s") * NC + lax.axis_index("c")
    base = wid * EPW
    pltpu.sync_copy(src_hbm.at[pl.ds(base, EPW)], src_all)
    pltpu.sync_copy(dst_hbm.at[pl.ds(base, EPW)], dst_all)
    lane16 = lax.iota(jnp.int32, LANES) * LANES
    sems = ((sem_p0, sem_q0), (sem_p1, sem_q1))

    def issue(g, b):
        src16 = src_all[pl.ds(g * LANES, LANES)]
        dst16 = dst_all[pl.ds(g * LANES, LANES)]
        pltpu.async_copy(p_hbm.at[src16], p_rows.at[b], sems[b][0])
        pltpu.async_copy(q_hbm.at[dst16], q_rows.at[b], sems[b][1])

    def drain(g, b):
        src16 = src_all[pl.ds(g * LANES, LANES)]
        dst16 = dst_all[pl.ds(g * LANES, LANES)]
        pltpu.make_async_copy(p_hbm.at[src16], p_rows.at[b], sems[b][0]).wait()
        pltpu.make_async_copy(q_hbm.at[dst16], q_rows.at[b], sems[b][1]).wait()

    def compute(g, b):
        for e in range(LANES):
            acc = (p_rows[b, e, pl.ds(0, LANES)]
                   * q_rows[b, e, pl.ds(0, LANES)])
            for j in range(1, DD // LANES):
                acc = acc + (p_rows[b, e, pl.ds(j * LANES, LANES)]
                             * q_rows[b, e, pl.ds(j * LANES, LANES)])
            accbuf[pl.ds(e * LANES, LANES)] = acc
        totals = plsc.load_gather(accbuf, [lane16])
        for j in range(1, LANES):
            totals = totals + plsc.load_gather(accbuf, [lane16 + j])
        out_all[pl.ds(g * LANES, LANES)] = 1.0 / (1.0 + jnp.exp(-0.5 * totals))

    issue(0, 0)

    def step(k, carry):
        g0 = 2 * k
        g1 = g0 + 1
        issue(g1, 1)
        drain(g0, 0)
        compute(g0, 0)
        issue(jnp.minimum(g0 + 2, G - 1), 0)
        drain(g1, 1)
        compute(g1, 1)
        return carry

    lax.fori_loop(0, G // 2, step, 0)
    drain(G - 1, 0)
    pltpu.sync_copy(out_all, out_hbm.at[pl.ds(base, EPW)])


def _edge_scores(P, Q, src, dst):
    mesh = plsc.VectorSubcoreMesh(core_axis_name="c", subcore_axis_name="s",
                                  num_cores=NC, num_subcores=NS)
    run = pl.kernel(
        _sc_edge_body,
        out_type=jax.ShapeDtypeStruct((E_PAD,), jnp.float32),
        mesh=mesh,
        compiler_params=pltpu.CompilerParams(needs_layout_passes=False),
        scratch_types=[
            pltpu.VMEM((EPW,), jnp.int32),
            pltpu.VMEM((EPW,), jnp.int32),
            pltpu.VMEM((EPW,), jnp.float32),
            pltpu.VMEM((2, LANES, DD), jnp.float32),
            pltpu.VMEM((2, LANES, DD), jnp.float32),
            pltpu.VMEM((LANES * LANES,), jnp.float32),
            pltpu.SemaphoreType.DMA,
            pltpu.SemaphoreType.DMA,
            pltpu.SemaphoreType.DMA,
            pltpu.SemaphoreType.DMA,
        ],
    )
    return run(P, Q, src, dst)


def kernel(z_in, z_out, z_self, edge_index, W_in, b_in, W_out, b_out):
    P, Q = _build_pq(z_in, z_out, z_self, W_in, b_in, W_out, b_out)
    src = jnp.pad(edge_index[0].astype(jnp.int32), (0, E_PAD - E))
    dst = jnp.pad(edge_index[1].astype(jnp.int32), (0, E_PAD - E))
    return _edge_scores(P, Q, src, dst)[:E]


# speedup vs baseline: 2.2983x; 1.1793x over previous
import jax
import jax.numpy as jnp
from jax import lax
from jax.experimental import pallas as pl
from jax.experimental.pallas import tpu as pltpu
from jax.experimental.pallas import tpu_sc as plsc

N = 10000
D = 256
E = 160000

NC = 2
NS = 16
NW = NC * NS
LANES = 16
DD = 2 * D
G = 314
EPW = G * LANES
E_PAD = NW * EPW

ROWS_TC = 1000


def _tc_pack_body(zs_ref, zo_ref, zi_ref, wi_ref, bi_ref, wo_ref, bo_ref,
                  p_ref, q_ref):
    zs = zs_ref[...]
    a_in = lax.dot_general(zs, wi_ref[...], (((1,), (1,)), ((), ())),
                           preferred_element_type=jnp.float32) + bi_ref[...]
    a_out = lax.dot_general(zs, wo_ref[...], (((1,), (1,)), ((), ())),
                            preferred_element_type=jnp.float32) + bo_ref[...]
    p_ref[:, :D] = zo_ref[...].astype(jnp.bfloat16)
    p_ref[:, D:] = a_in.astype(jnp.bfloat16)
    q_ref[:, :D] = a_out.astype(jnp.bfloat16)
    q_ref[:, D:] = zi_ref[...].astype(jnp.bfloat16)


def _build_pq(z_in, z_out, z_self, W_in, b_in, W_out, b_out):
    b_in2 = b_in.reshape(1, D)
    b_out2 = b_out.reshape(1, D)
    grid = (N // ROWS_TC,)
    row_spec = pl.BlockSpec((ROWS_TC, D), lambda i: (i, 0))
    full_spec = pl.BlockSpec((D, D), lambda i: (0, 0))
    bias_spec = pl.BlockSpec((1, D), lambda i: (0, 0))
    out_spec = pl.BlockSpec((ROWS_TC, DD), lambda i: (i, 0))
    return pl.pallas_call(
        _tc_pack_body,
        grid=grid,
        in_specs=[row_spec, row_spec, row_spec, full_spec, bias_spec,
                  full_spec, bias_spec],
        out_specs=[out_spec, out_spec],
        out_shape=[jax.ShapeDtypeStruct((N, DD), jnp.bfloat16),
                   jax.ShapeDtypeStruct((N, DD), jnp.bfloat16)],
    )(z_self, z_out, z_in, W_in, b_in2, W_out, b_out2)


def _sc_edge_body(p_hbm, q_hbm, src_hbm, dst_hbm, out_hbm,
                  src_all, dst_all, out_all, p_rows, q_rows, accbuf,
                  sem_p0, sem_q0, sem_p1, sem_q1):
    wid = lax.axis_index("s") * NC + lax.axis_index("c")
    base = wid * EPW
    pltpu.sync_copy(src_hbm.at[pl.ds(base, EPW)], src_all)
    pltpu.sync_copy(dst_hbm.at[pl.ds(base, EPW)], dst_all)
    lane16 = lax.iota(jnp.int32, LANES) * LANES
    sems = ((sem_p0, sem_q0), (sem_p1, sem_q1))

    def issue(g, b):
        src16 = src_all[pl.ds(g * LANES, LANES)]
        dst16 = dst_all[pl.ds(g * LANES, LANES)]
        pltpu.async_copy(p_hbm.at[src16], p_rows.at[b], sems[b][0])
        pltpu.async_copy(q_hbm.at[dst16], q_rows.at[b], sems[b][1])

    def drain(g, b):
        src16 = src_all[pl.ds(g * LANES, LANES)]
        dst16 = dst_all[pl.ds(g * LANES, LANES)]
        pltpu.make_async_copy(p_hbm.at[src16], p_rows.at[b], sems[b][0]).wait()
        pltpu.make_async_copy(q_hbm.at[dst16], q_rows.at[b], sems[b][1]).wait()

    def compute(g, b):
        for e in range(LANES):
            acc = jnp.zeros((LANES,), jnp.float32)
            for j in range(DD // (2 * LANES)):
                pw = p_rows[b, e, pl.ds(j * LANES, LANES)]
                qw = q_rows[b, e, pl.ds(j * LANES, LANES)]
                p32 = plsc.bitcast(pw, jnp.bfloat16)
                q32 = plsc.bitcast(qw, jnp.bfloat16)
                pa, pb = plsc.unpack(p32, format=plsc.PackFormat.INTERLEAVED)
                qa, qb = plsc.unpack(q32, format=plsc.PackFormat.INTERLEAVED)
                acc = acc + pa * qa + pb * qb
            accbuf[pl.ds(e * LANES, LANES)] = acc
        totals = plsc.load_gather(accbuf, [lane16])
        for j in range(1, LANES):
            totals = totals + plsc.load_gather(accbuf, [lane16 + j])
        out_all[pl.ds(g * LANES, LANES)] = 1.0 / (1.0 + jnp.exp(-0.5 * totals))

    issue(0, 0)

    def step(k, carry):
        g0 = 2 * k
        g1 = g0 + 1
        issue(g1, 1)
        drain(g0, 0)
        compute(g0, 0)
        issue(jnp.minimum(g0 + 2, G - 1), 0)
        drain(g1, 1)
        compute(g1, 1)
        return carry

    lax.fori_loop(0, G // 2, step, 0)
    drain(G - 1, 0)
    pltpu.sync_copy(out_all, out_hbm.at[pl.ds(base, EPW)])


def _edge_scores(P, Q, src, dst):
    mesh = plsc.VectorSubcoreMesh(core_axis_name="c", subcore_axis_name="s",
                                  num_cores=NC, num_subcores=NS)
    run = pl.kernel(
        _sc_edge_body,
        out_type=jax.ShapeDtypeStruct((E_PAD,), jnp.float32),
        mesh=mesh,
        compiler_params=pltpu.CompilerParams(needs_layout_passes=False),
        scratch_types=[
            pltpu.VMEM((EPW,), jnp.int32),
            pltpu.VMEM((EPW,), jnp.int32),
            pltpu.VMEM((EPW,), jnp.float32),
            pltpu.VMEM((2, LANES, DD // 2), jnp.int32),
            pltpu.VMEM((2, LANES, DD // 2), jnp.int32),
            pltpu.VMEM((LANES * LANES,), jnp.float32),
            pltpu.SemaphoreType.DMA,
            pltpu.SemaphoreType.DMA,
            pltpu.SemaphoreType.DMA,
            pltpu.SemaphoreType.DMA,
        ],
    )
    return run(P, Q, src, dst)


def kernel(z_in, z_out, z_self, edge_index, W_in, b_in, W_out, b_out):
    P, Q = _build_pq(z_in, z_out, z_self, W_in, b_in, W_out, b_out)
    P32 = lax.bitcast_convert_type(P.reshape(N, DD // 2, 2), jnp.int32)
    Q32 = lax.bitcast_convert_type(Q.reshape(N, DD // 2, 2), jnp.int32)
    src = jnp.pad(edge_index[0].astype(jnp.int32), (0, E_PAD - E))
    dst = jnp.pad(edge_index[1].astype(jnp.int32), (0, E_PAD - E))
    return _edge_scores(P32, Q32, src, dst)[:E]


# speedup vs baseline: 4.5677x; 1.9874x over previous
import jax
import jax.numpy as jnp
from jax import lax
from jax.experimental import pallas as pl
from jax.experimental.pallas import tpu as pltpu
from jax.experimental.pallas import tpu_sc as plsc

N = 10000
D = 256
E = 160000

NC = 2
NS = 16
NW = NC * NS
LANES = 16
DD = 2 * D
G = 314
EPW = G * LANES
E_PAD = NW * EPW

ROWS_TC = 1000


def _pack2(lo_f32, hi_f32):
    lo = lax.bitcast_convert_type(lo_f32.astype(jnp.bfloat16), jnp.uint16)
    hi = lax.bitcast_convert_type(hi_f32.astype(jnp.bfloat16), jnp.uint16)
    word = lo.astype(jnp.uint32) | (hi.astype(jnp.uint32) << 16)
    return lax.bitcast_convert_type(word, jnp.int32)


def _tc_pack_body(zs_ref, zo_ref, zi_ref, wi_ref, bi_ref, wo_ref, bo_ref,
                  p_ref, q_ref):
    zs = zs_ref[...]
    a_in = lax.dot_general(zs, wi_ref[...], (((1,), (1,)), ((), ())),
                           preferred_element_type=jnp.float32) + bi_ref[...]
    a_out = lax.dot_general(zs, wo_ref[...], (((1,), (1,)), ((), ())),
                            preferred_element_type=jnp.float32) + bo_ref[...]
    p_ref[...] = _pack2(zo_ref[...], a_in)
    q_ref[...] = _pack2(a_out, zi_ref[...])


def _build_pq(z_in, z_out, z_self, W_in, b_in, W_out, b_out):
    b_in2 = b_in.reshape(1, D)
    b_out2 = b_out.reshape(1, D)
    grid = (N // ROWS_TC,)
    row_spec = pl.BlockSpec((ROWS_TC, D), lambda i: (i, 0))
    full_spec = pl.BlockSpec((D, D), lambda i: (0, 0))
    bias_spec = pl.BlockSpec((1, D), lambda i: (0, 0))
    out_spec = pl.BlockSpec((ROWS_TC, D), lambda i: (i, 0))
    return pl.pallas_call(
        _tc_pack_body,
        grid=grid,
        in_specs=[row_spec, row_spec, row_spec, full_spec, bias_spec,
                  full_spec, bias_spec],
        out_specs=[out_spec, out_spec],
        out_shape=[jax.ShapeDtypeStruct((N, D), jnp.int32),
                   jax.ShapeDtypeStruct((N, D), jnp.int32)],
    )(z_self, z_out, z_in, W_in, b_in2, W_out, b_out2)


def _sc_edge_body(p_hbm, q_hbm, src_hbm, dst_hbm, out_hbm,
                  src_all, dst_all, out_all, p_rows, q_rows, accbuf,
                  sem_p0, sem_q0, sem_p1, sem_q1):
    wid = lax.axis_index("s") * NC + lax.axis_index("c")
    base = wid * EPW
    pltpu.sync_copy(src_hbm.at[pl.ds(base, EPW)], src_all)
    pltpu.sync_copy(dst_hbm.at[pl.ds(base, EPW)], dst_all)
    lane16 = lax.iota(jnp.int32, LANES) * LANES
    sems = ((sem_p0, sem_q0), (sem_p1, sem_q1))

    def issue(g, b):
        src16 = src_all[pl.ds(g * LANES, LANES)]
        dst16 = dst_all[pl.ds(g * LANES, LANES)]
        pltpu.async_copy(p_hbm.at[src16], p_rows.at[b], sems[b][0])
        pltpu.async_copy(q_hbm.at[dst16], q_rows.at[b], sems[b][1])

    def drain(g, b):
        src16 = src_all[pl.ds(g * LANES, LANES)]
        dst16 = dst_all[pl.ds(g * LANES, LANES)]
        pltpu.make_async_copy(p_hbm.at[src16], p_rows.at[b], sems[b][0]).wait()
        pltpu.make_async_copy(q_hbm.at[dst16], q_rows.at[b], sems[b][1]).wait()

    def compute(g, b):
        for e in range(LANES):
            acc = jnp.zeros((LANES,), jnp.float32)
            for j in range(DD // (2 * LANES)):
                pw = p_rows[b, e, pl.ds(j * LANES, LANES)]
                qw = q_rows[b, e, pl.ds(j * LANES, LANES)]
                p32 = plsc.bitcast(pw, jnp.bfloat16)
                q32 = plsc.bitcast(qw, jnp.bfloat16)
                pa, pb = plsc.unpack(p32, format=plsc.PackFormat.INTERLEAVED)
                qa, qb = plsc.unpack(q32, format=plsc.PackFormat.INTERLEAVED)
                acc = acc + pa * qa + pb * qb
            accbuf[pl.ds(e * LANES, LANES)] = acc
        totals = plsc.load_gather(accbuf, [lane16])
        for j in range(1, LANES):
            totals = totals + plsc.load_gather(accbuf, [lane16 + j])
        out_all[pl.ds(g * LANES, LANES)] = 1.0 / (1.0 + jnp.exp(-0.5 * totals))

    issue(0, 0)

    def step(k, carry):
        g0 = 2 * k
        g1 = g0 + 1
        issue(g1, 1)
        drain(g0, 0)
        compute(g0, 0)
        issue(jnp.minimum(g0 + 2, G - 1), 0)
        drain(g1, 1)
        compute(g1, 1)
        return carry

    lax.fori_loop(0, G // 2, step, 0)
    drain(G - 1, 0)
    pltpu.sync_copy(out_all, out_hbm.at[pl.ds(base, EPW)])


def _edge_scores(P, Q, src, dst):
    mesh = plsc.VectorSubcoreMesh(core_axis_name="c", subcore_axis_name="s",
                                  num_cores=NC, num_subcores=NS)
    run = pl.kernel(
        _sc_edge_body,
        out_type=jax.ShapeDtypeStruct((E_PAD,), jnp.float32),
        mesh=mesh,
        compiler_params=pltpu.CompilerParams(needs_layout_passes=False),
        scratch_types=[
            pltpu.VMEM((EPW,), jnp.int32),
            pltpu.VMEM((EPW,), jnp.int32),
            pltpu.VMEM((EPW,), jnp.float32),
            pltpu.VMEM((2, LANES, DD // 2), jnp.int32),
            pltpu.VMEM((2, LANES, DD // 2), jnp.int32),
            pltpu.VMEM((LANES * LANES,), jnp.float32),
            pltpu.SemaphoreType.DMA,
            pltpu.SemaphoreType.DMA,
            pltpu.SemaphoreType.DMA,
            pltpu.SemaphoreType.DMA,
        ],
    )
    return run(P, Q, src, dst)


def kernel(z_in, z_out, z_self, edge_index, W_in, b_in, W_out, b_out):
    P32, Q32 = _build_pq(z_in, z_out, z_self, W_in, b_in, W_out, b_out)
    src = jnp.pad(edge_index[0].astype(jnp.int32), (0, E_PAD - E))
    dst = jnp.pad(edge_index[1].astype(jnp.int32), (0, E_PAD - E))
    return _edge_scores(P32, Q32, src, dst)[:E]


# speedup vs baseline: 5.5935x; 1.2246x over previous
import jax
import jax.numpy as jnp
from jax import lax
from jax.experimental import pallas as pl
from jax.experimental.pallas import tpu as pltpu
from jax.experimental.pallas import tpu_sc as plsc

N = 10000
D = 256
E = 160000

NC = 2
NS = 16
NW = NC * NS
LANES = 16
DD = 2 * D
G = 316
SLOTS = 4
EPW = G * LANES
E_PAD = NW * EPW

ROWS_TC = 1000


def _pack2(lo_f32, hi_f32):
    lo = lax.bitcast_convert_type(lo_f32.astype(jnp.bfloat16), jnp.uint16)
    hi = lax.bitcast_convert_type(hi_f32.astype(jnp.bfloat16), jnp.uint16)
    word = lo.astype(jnp.uint32) | (hi.astype(jnp.uint32) << 16)
    return lax.bitcast_convert_type(word, jnp.int32)


def _tc_pack_body(zs_ref, zo_ref, zi_ref, wi_ref, bi_ref, wo_ref, bo_ref,
                  p_ref, q_ref):
    zs = zs_ref[...]
    a_in = lax.dot_general(zs, wi_ref[...], (((1,), (1,)), ((), ())),
                           preferred_element_type=jnp.float32) + bi_ref[...]
    a_out = lax.dot_general(zs, wo_ref[...], (((1,), (1,)), ((), ())),
                            preferred_element_type=jnp.float32) + bo_ref[...]
    p_ref[...] = _pack2(zo_ref[...], a_in)
    q_ref[...] = _pack2(a_out, zi_ref[...])


def _build_pq(z_in, z_out, z_self, W_in, b_in, W_out, b_out):
    b_in2 = b_in.reshape(1, D)
    b_out2 = b_out.reshape(1, D)
    grid = (N // ROWS_TC,)
    row_spec = pl.BlockSpec((ROWS_TC, D), lambda i: (i, 0))
    full_spec = pl.BlockSpec((D, D), lambda i: (0, 0))
    bias_spec = pl.BlockSpec((1, D), lambda i: (0, 0))
    out_spec = pl.BlockSpec((ROWS_TC, D), lambda i: (i, 0))
    return pl.pallas_call(
        _tc_pack_body,
        grid=grid,
        in_specs=[row_spec, row_spec, row_spec, full_spec, bias_spec,
                  full_spec, bias_spec],
        out_specs=[out_spec, out_spec],
        out_shape=[jax.ShapeDtypeStruct((N, D), jnp.int32),
                   jax.ShapeDtypeStruct((N, D), jnp.int32)],
    )(z_self, z_out, z_in, W_in, b_in2, W_out, b_out2)


def _sc_edge_body(p_hbm, q_hbm, src_hbm, dst_hbm, out_hbm,
                  src_all, dst_all, out_all, p_rows, q_rows, accbuf,
                  *sems):
    wid = lax.axis_index("s") * NC + lax.axis_index("c")
    base = wid * EPW
    pltpu.sync_copy(src_hbm.at[pl.ds(base, EPW)], src_all)
    pltpu.sync_copy(dst_hbm.at[pl.ds(base, EPW)], dst_all)
    lane16 = lax.iota(jnp.int32, LANES) * LANES

    def issue(g, b):
        src16 = src_all[pl.ds(g * LANES, LANES)]
        dst16 = dst_all[pl.ds(g * LANES, LANES)]
        pltpu.async_copy(p_hbm.at[src16], p_rows.at[b], sems[2 * b])
        pltpu.async_copy(q_hbm.at[dst16], q_rows.at[b], sems[2 * b + 1])

    def drain(g, b):
        src16 = src_all[pl.ds(g * LANES, LANES)]
        dst16 = dst_all[pl.ds(g * LANES, LANES)]
        pltpu.make_async_copy(p_hbm.at[src16], p_rows.at[b],
                              sems[2 * b]).wait()
        pltpu.make_async_copy(q_hbm.at[dst16], q_rows.at[b],
                              sems[2 * b + 1]).wait()

    def compute(g, b):
        def edge(e, carry):
            acc = jnp.zeros((LANES,), jnp.float32)
            for j in range(DD // (2 * LANES)):
                pw = p_rows[b, e, pl.ds(j * LANES, LANES)]
                qw = q_rows[b, e, pl.ds(j * LANES, LANES)]
                p32 = plsc.bitcast(pw, jnp.bfloat16)
                q32 = plsc.bitcast(qw, jnp.bfloat16)
                pa, pb = plsc.unpack(p32, format=plsc.PackFormat.INTERLEAVED)
                qa, qb = plsc.unpack(q32, format=plsc.PackFormat.INTERLEAVED)
                acc = acc + pa * qa + pb * qb
            accbuf[pl.ds(e * LANES, LANES)] = acc
            return carry

        lax.fori_loop(0, LANES, edge, 0)
        totals = plsc.load_gather(accbuf, [lane16])
        for j in range(1, LANES):
            totals = totals + plsc.load_gather(accbuf, [lane16 + j])
        out_all[pl.ds(g * LANES, LANES)] = 1.0 / (1.0 + jnp.exp(-0.5 * totals))

    for b in range(SLOTS - 1):
        issue(b, b)

    def step(k, carry):
        for b in range(SLOTS):
            g = SLOTS * k + b
            drain(g, b)
            issue(jnp.minimum(g + SLOTS - 1, G - 1), (b + SLOTS - 1) % SLOTS)
            compute(g, b)
        return carry

    lax.fori_loop(0, G // SLOTS, step, 0)
    for b in range(SLOTS - 1):
        drain(G - 1, b)
    pltpu.sync_copy(out_all, out_hbm.at[pl.ds(base, EPW)])


def _edge_scores(P, Q, src, dst):
    mesh = plsc.VectorSubcoreMesh(core_axis_name="c", subcore_axis_name="s",
                                  num_cores=NC, num_subcores=NS)
    run = pl.kernel(
        _sc_edge_body,
        out_type=jax.ShapeDtypeStruct((E_PAD,), jnp.float32),
        mesh=mesh,
        compiler_params=pltpu.CompilerParams(needs_layout_passes=False),
        scratch_types=[
            pltpu.VMEM((EPW,), jnp.int32),
            pltpu.VMEM((EPW,), jnp.int32),
            pltpu.VMEM((EPW,), jnp.float32),
            pltpu.VMEM((SLOTS, LANES, DD // 2), jnp.int32),
            pltpu.VMEM((SLOTS, LANES, DD // 2), jnp.int32),
            pltpu.VMEM((LANES * LANES,), jnp.float32),
        ] + [pltpu.SemaphoreType.DMA] * (2 * SLOTS),
    )
    return run(P, Q, src, dst)


def kernel(z_in, z_out, z_self, edge_index, W_in, b_in, W_out, b_out):
    P32, Q32 = _build_pq(z_in, z_out, z_self, W_in, b_in, W_out, b_out)
    src = jnp.pad(edge_index[0].astype(jnp.int32), (0, E_PAD - E))
    dst = jnp.pad(edge_index[1].astype(jnp.int32), (0, E_PAD - E))
    return _edge_scores(P32, Q32, src, dst)[:E]


# speedup vs baseline: 5.8420x; 1.0444x over previous
import jax
import jax.numpy as jnp
from jax import lax
from jax.experimental import pallas as pl
from jax.experimental.pallas import tpu as pltpu
from jax.experimental.pallas import tpu_sc as plsc

N = 10000
D = 256
E = 160000

NC = 2
NS = 16
NW = NC * NS
LANES = 16
DD = 2 * D
SLOTS = 4
CH = 32
NCH = 160
EPW = NCH * CH

ROWS_TC = 1000


def _pack2(lo_f32, hi_f32):
    lo = lax.bitcast_convert_type(lo_f32.astype(jnp.bfloat16), jnp.uint16)
    hi = lax.bitcast_convert_type(hi_f32.astype(jnp.bfloat16), jnp.uint16)
    word = lo.astype(jnp.uint32) | (hi.astype(jnp.uint32) << 16)
    return lax.bitcast_convert_type(word, jnp.int32)


def _tc_pack_body(zs_ref, zo_ref, zi_ref, wi_ref, bi_ref, wo_ref, bo_ref,
                  p_ref, q_ref):
    zs = zs_ref[...]
    a_in = lax.dot_general(zs, wi_ref[...], (((1,), (1,)), ((), ())),
                           preferred_element_type=jnp.float32) + bi_ref[...]
    a_out = lax.dot_general(zs, wo_ref[...], (((1,), (1,)), ((), ())),
                            preferred_element_type=jnp.float32) + bo_ref[...]
    p_ref[...] = _pack2(zo_ref[...], a_in)
    q_ref[...] = _pack2(a_out, zi_ref[...])


def _build_pq(z_in, z_out, z_self, W_in, b_in, W_out, b_out):
    b_in2 = b_in.reshape(1, D)
    b_out2 = b_out.reshape(1, D)
    grid = (N // ROWS_TC,)
    row_spec = pl.BlockSpec((ROWS_TC, D), lambda i: (i, 0))
    full_spec = pl.BlockSpec((D, D), lambda i: (0, 0))
    bias_spec = pl.BlockSpec((1, D), lambda i: (0, 0))
    out_spec = pl.BlockSpec((ROWS_TC, D), lambda i: (i, 0))
    return pl.pallas_call(
        _tc_pack_body,
        grid=grid,
        in_specs=[row_spec, row_spec, row_spec, full_spec, bias_spec,
                  full_spec, bias_spec],
        out_specs=[out_spec, out_spec],
        out_shape=[jax.ShapeDtypeStruct((N, D), jnp.int32),
                   jax.ShapeDtypeStruct((N, D), jnp.int32)],
    )(z_self, z_out, z_in, W_in, b_in2, W_out, b_out2)


def _sc_edge_body(p_hbm, q_hbm, ei_hbm, out_hbm,
                  src_all, dst_all, out_all, p_rows, q_rows, accbuf,
                  *sems):
    wid = lax.axis_index("s") * NC + lax.axis_index("c")
    base = jnp.minimum(wid * EPW, E - EPW)
    pltpu.sync_copy(ei_hbm.at[0, pl.ds(base, EPW)], src_all)
    pltpu.sync_copy(ei_hbm.at[1, pl.ds(base, EPW)], dst_all)
    lane16 = lax.iota(jnp.int32, LANES) * LANES

    def issue(c, b):
        pltpu.async_copy(p_hbm.at[src_all.at[pl.ds(c * CH, CH)]],
                         p_rows.at[b], sems[2 * b])
        pltpu.async_copy(q_hbm.at[dst_all.at[pl.ds(c * CH, CH)]],
                         q_rows.at[b], sems[2 * b + 1])

    def drain(c, b):
        pltpu.make_async_copy(p_hbm.at[src_all.at[pl.ds(c * CH, CH)]],
                              p_rows.at[b], sems[2 * b]).wait()
        pltpu.make_async_copy(q_hbm.at[dst_all.at[pl.ds(c * CH, CH)]],
                              q_rows.at[b], sems[2 * b + 1]).wait()

    def compute(c, b):
        for sub in range(CH // LANES):
            def edge(e, carry):
                r = sub * LANES + e
                acc = jnp.zeros((LANES,), jnp.float32)
                for j in range(DD // (2 * LANES)):
                    pw = p_rows[b, r, pl.ds(j * LANES, LANES)]
                    qw = q_rows[b, r, pl.ds(j * LANES, LANES)]
                    prod = (plsc.bitcast(pw, jnp.bfloat16)
                            * plsc.bitcast(qw, jnp.bfloat16))
                    pa, pb = plsc.unpack(prod,
                                         format=plsc.PackFormat.INTERLEAVED)
                    acc = acc + pa + pb
                accbuf[pl.ds(e * LANES, LANES)] = acc
                return carry

            lax.fori_loop(0, LANES, edge, 0)
            totals = plsc.load_gather(accbuf, [lane16])
            for j in range(1, LANES):
                totals = totals + plsc.load_gather(accbuf, [lane16 + j])
            out_all[pl.ds(c * CH + sub * LANES, LANES)] = (
                1.0 / (1.0 + jnp.exp(-0.5 * totals)))

    for b in range(SLOTS - 1):
        issue(b, b)

    def step(k, carry):
        for b in range(SLOTS):
            c = SLOTS * k + b
            drain(c, b)
            issue(jnp.minimum(c + SLOTS - 1, NCH - 1), (b + SLOTS - 1) % SLOTS)
            compute(c, b)
        return carry

    lax.fori_loop(0, NCH // SLOTS, step, 0)
    for b in range(SLOTS - 1):
        drain(NCH - 1, b)
    pltpu.sync_copy(out_all, out_hbm.at[pl.ds(base, EPW)])


def _edge_scores(P, Q, edge_index):
    mesh = plsc.VectorSubcoreMesh(core_axis_name="c", subcore_axis_name="s",
                                  num_cores=NC, num_subcores=NS)
    run = pl.kernel(
        _sc_edge_body,
        out_type=jax.ShapeDtypeStruct((E,), jnp.float32),
        mesh=mesh,
        compiler_params=pltpu.CompilerParams(needs_layout_passes=False),
        scratch_types=[
            pltpu.VMEM((EPW,), jnp.int32),
            pltpu.VMEM((EPW,), jnp.int32),
            pltpu.VMEM((EPW,), jnp.float32),
            pltpu.VMEM((SLOTS, CH, DD // 2), jnp.int32),
            pltpu.VMEM((SLOTS, CH, DD // 2), jnp.int32),
            pltpu.VMEM((LANES * LANES,), jnp.float32),
        ] + [pltpu.SemaphoreType.DMA] * (2 * SLOTS),
    )
    return run(P, Q, edge_index)


def kernel(z_in, z_out, z_self, edge_index, W_in, b_in, W_out, b_out):
    P32, Q32 = _build_pq(z_in, z_out, z_self, W_in, b_in, W_out, b_out)
    return _edge_scores(P32, Q32, edge_index.astype(jnp.int32))


# speedup vs baseline: 8.3487x; 1.4291x over previous
import jax
import jax.numpy as jnp
from jax import lax
from jax.experimental import pallas as pl
from jax.experimental.pallas import tpu as pltpu
from jax.experimental.pallas import tpu_sc as plsc

N = 10000
D = 256
E = 160000

NC = 2
NS = 16
NW = NC * NS
LANES = 16
DD = 2 * D
SLOTS = 4
CH = 32
NCH = 160
EPW = NCH * CH

ROWS_TC = 1000


def _pack2(lo_f32, hi_f32):
    lo = lax.bitcast_convert_type(lo_f32.astype(jnp.bfloat16), jnp.uint16)
    hi = lax.bitcast_convert_type(hi_f32.astype(jnp.bfloat16), jnp.uint16)
    word = lo.astype(jnp.uint32) | (hi.astype(jnp.uint32) << 16)
    return lax.bitcast_convert_type(word, jnp.int32)


def _tc_pack_body(zs_ref, zo_ref, zi_ref, wi_ref, bi_ref, wo_ref, bo_ref,
                  p_ref, q_ref):
    zs = zs_ref[...]
    a_in = lax.dot_general(zs, wi_ref[...], (((1,), (1,)), ((), ())),
                           preferred_element_type=jnp.float32) + bi_ref[...]
    a_out = lax.dot_general(zs, wo_ref[...], (((1,), (1,)), ((), ())),
                            preferred_element_type=jnp.float32) + bo_ref[...]
    p_ref[...] = _pack2(zo_ref[...], a_in)
    q_ref[...] = _pack2(a_out, zi_ref[...])


def _build_pq(z_in, z_out, z_self, W_in, b_in, W_out, b_out):
    b_in2 = b_in.reshape(1, D)
    b_out2 = b_out.reshape(1, D)
    grid = (N // ROWS_TC,)
    row_spec = pl.BlockSpec((ROWS_TC, D), lambda i: (i, 0))
    full_spec = pl.BlockSpec((D, D), lambda i: (0, 0))
    bias_spec = pl.BlockSpec((1, D), lambda i: (0, 0))
    out_spec = pl.BlockSpec((ROWS_TC, D), lambda i: (i, 0))
    return pl.pallas_call(
        _tc_pack_body,
        grid=grid,
        in_specs=[row_spec, row_spec, row_spec, full_spec, bias_spec,
                  full_spec, bias_spec],
        out_specs=[out_spec, out_spec],
        out_shape=[jax.ShapeDtypeStruct((N, D), jnp.int32),
                   jax.ShapeDtypeStruct((N, D), jnp.int32)],
    )(z_self, z_out, z_in, W_in, b_in2, W_out, b_out2)


def _sc_edge_body(p_hbm, q_hbm, ei_hbm, out_hbm,
                  src_all, dst_all, out_all, p_rows, q_rows, accbuf,
                  *sems):
    wid = lax.axis_index("s") * NC + lax.axis_index("c")
    base = jnp.minimum(wid * EPW, E - EPW)
    pltpu.sync_copy(ei_hbm.at[0, pl.ds(base, EPW)], src_all)
    pltpu.sync_copy(ei_hbm.at[1, pl.ds(base, EPW)], dst_all)
    lane16 = lax.iota(jnp.int32, LANES) * LANES

    def issue(c, b):
        pltpu.async_copy(p_hbm.at[src_all.at[pl.ds(c * CH, CH)]],
                         p_rows.at[b], sems[2 * b])
        pltpu.async_copy(q_hbm.at[dst_all.at[pl.ds(c * CH, CH)]],
                         q_rows.at[b], sems[2 * b + 1])

    def drain(c, b):
        pltpu.make_async_copy(p_hbm.at[src_all.at[pl.ds(c * CH, CH)]],
                              p_rows.at[b], sems[2 * b]).wait()
        pltpu.make_async_copy(q_hbm.at[dst_all.at[pl.ds(c * CH, CH)]],
                              q_rows.at[b], sems[2 * b + 1]).wait()

    def compute(c, b):
        for sub in range(CH // LANES):
            @plsc.parallel_loop(0, LANES, step=1, unroll=2)
            def edge(e):
                r = sub * LANES + e
                acc0 = jnp.zeros((LANES,), jnp.float32)
                acc1 = jnp.zeros((LANES,), jnp.float32)
                for j in range(DD // (2 * LANES)):
                    pw = p_rows[b, r, pl.ds(j * LANES, LANES)]
                    qw = q_rows[b, r, pl.ds(j * LANES, LANES)]
                    prod = (plsc.bitcast(pw, jnp.bfloat16)
                            * plsc.bitcast(qw, jnp.bfloat16))
                    pa, pb = plsc.unpack(prod,
                                         format=plsc.PackFormat.INTERLEAVED)
                    if j % 2 == 0:
                        acc0 = acc0 + (pa + pb)
                    else:
                        acc1 = acc1 + (pa + pb)
                accbuf[pl.ds(e * LANES, LANES)] = acc0 + acc1
            totals = plsc.load_gather(accbuf, [lane16])
            for j in range(1, LANES):
                totals = totals + plsc.load_gather(accbuf, [lane16 + j])
            out_all[pl.ds(c * CH + sub * LANES, LANES)] = (
                1.0 / (1.0 + jnp.exp(-0.5 * totals)))

    for b in range(SLOTS - 1):
        issue(b, b)

    def step(k, carry):
        for b in range(SLOTS):
            c = SLOTS * k + b
            drain(c, b)
            issue(jnp.minimum(c + SLOTS - 1, NCH - 1), (b + SLOTS - 1) % SLOTS)
            compute(c, b)
        return carry

    lax.fori_loop(0, NCH // SLOTS, step, 0)
    for b in range(SLOTS - 1):
        drain(NCH - 1, b)
    pltpu.sync_copy(out_all, out_hbm.at[pl.ds(base, EPW)])


def _edge_scores(P, Q, edge_index):
    mesh = plsc.VectorSubcoreMesh(core_axis_name="c", subcore_axis_name="s",
                                  num_cores=NC, num_subcores=NS)
    run = pl.kernel(
        _sc_edge_body,
        out_type=jax.ShapeDtypeStruct((E,), jnp.float32),
        mesh=mesh,
        compiler_params=pltpu.CompilerParams(needs_layout_passes=False),
        scratch_types=[
            pltpu.VMEM((EPW,), jnp.int32),
            pltpu.VMEM((EPW,), jnp.int32),
            pltpu.VMEM((EPW,), jnp.float32),
            pltpu.VMEM((SLOTS, CH, DD // 2), jnp.int32),
            pltpu.VMEM((SLOTS, CH, DD // 2), jnp.int32),
            pltpu.VMEM((LANES * LANES,), jnp.float32),
        ] + [pltpu.SemaphoreType.DMA] * (2 * SLOTS),
    )
    return run(P, Q, edge_index)


def kernel(z_in, z_out, z_self, edge_index, W_in, b_in, W_out, b_out):
    P32, Q32 = _build_pq(z_in, z_out, z_self, W_in, b_in, W_out, b_out)
    return _edge_scores(P32, Q32, edge_index.astype(jnp.int32))
